# baseline (speedup 1.0000x reference)
import functools

import jax
import jax.numpy as jnp
from jax import lax
from jax.experimental import pallas as pl
from jax.experimental.pallas import tpu as pltpu
from jax.experimental.pallas import tpu_sc as plsc

N = 10000
E = 320000
F = 128

NC = 2
NS = 16
NW = NC * NS
EPW = E // NW
CH = 80
NCH = EPW // CH
RPT = 624
TAIL = N - NS * RPT
ZR = 16
LN = 16


def _mm_body(x_ref, w_ref, o_ref):
    o_ref[...] = jnp.dot(x_ref[...], w_ref[...],
                         preferred_element_type=jnp.float32)


def _matmul(x, W):
    blk = 1000
    return pl.pallas_call(
        _mm_body,
        grid=(N // blk,),
        in_specs=[
            pl.BlockSpec((blk, F), lambda i: (i, 0)),
            pl.BlockSpec((F, F), lambda i: (0, 0)),
        ],
        out_specs=pl.BlockSpec((blk, F), lambda i: (i, 0)),
        out_shape=jax.ShapeDtypeStruct((N, F), jnp.float32),
    )(x, W)


def _scale_chunk(rows, adjb):
    for go in range(0, CH, LN):
        a16 = adjb[pl.ds(go, LN)]
        for j in range(LN):
            for s in range(F // LN):
                sl = pl.ds(s * LN, LN)
                rows[go + j, sl] = rows[go + j, sl] * a16[j]


def _sc_body(support_hbm, src_hbm, dst_hbm, adj_hbm, out_hbm,
             dst_v, rows0_v, rows1_v, rows2_v,
             srcb0_v, srcb1_v, srcb2_v, adjb0_v, adjb1_v, adjb2_v,
             zbuf_v, acc_sh,
             gsem0, gsem1, gsem2, ssem0, ssem1, ssem2, isem0, isem1, isem2):
    cid = lax.axis_index("c")
    sid = lax.axis_index("s")
    wid = cid * NS + sid
    ebase = wid * EPW

    pltpu.sync_copy(dst_hbm.at[wid], dst_v)

    zeros = jnp.zeros((LN,), jnp.float32)

    def _zrow(i, carry):
        for s in range(F // LN):
            zbuf_v[i, pl.ds(s * LN, LN)] = zeros
        return carry

    lax.fori_loop(0, ZR, _zrow, 0)
    for r in range(RPT // ZR):
        pltpu.sync_copy(zbuf_v, acc_sh.at[pl.ds(sid * RPT + r * ZR, ZR)])

    @pl.when(sid == 0)
    def _zero_tail():
        pltpu.sync_copy(zbuf_v.at[pl.ds(0, TAIL)],
                        acc_sh.at[pl.ds(NS * RPT, TAIL)])

    plsc.subcore_barrier()

    rows = (rows0_v, rows1_v, rows2_v)
    srcb = (srcb0_v, srcb1_v, srcb2_v)
    adjb = (adjb0_v, adjb1_v, adjb2_v)
    gsem = (gsem0, gsem1, gsem2)
    ssem = (ssem0, ssem1, ssem2)
    isem = (isem0, isem1, isem2)

    def idx_issue(c, B):
        pltpu.async_copy(src_hbm.at[pl.ds(ebase + c * CH, CH)], srcb[B],
                         isem[B])
        pltpu.async_copy(adj_hbm.at[pl.ds(ebase + c * CH, CH)], adjb[B],
                         isem[B])

    def idx_wait(B):
        pltpu.make_async_copy(src_hbm.at[pl.ds(0, CH)], srcb[B],
                              isem[B]).wait()
        pltpu.make_async_copy(adj_hbm.at[pl.ds(0, CH)], adjb[B],
                              isem[B]).wait()

    def gather_issue(B):
        pltpu.async_copy(support_hbm.at[srcb[B]], rows[B], gsem[B])

    def gather_wait(B):
        pltpu.make_async_copy(support_hbm.at[pl.ds(0, CH)], rows[B],
                              gsem[B]).wait()

    def scatter_issue(c, B):
        pltpu.async_copy(rows[B], acc_sh.at[dst_v.at[c]], ssem[B], add=True)

    def scatter_wait(B):
        pltpu.make_async_copy(support_hbm.at[pl.ds(0, CH)], rows[B],
                              ssem[B]).wait()

    idx_issue(0, 0)
    idx_issue(1, 1)
    idx_issue(2, 2)
    idx_wait(0)
    gather_issue(0)
    idx_wait(1)
    gather_issue(1)

    NT = (NCH - 2) // 3

    def _triple(p, carry):
        for b in range(3):
            c = 3 * p + b
            B = b
            gather_wait(B)
            _scale_chunk(rows[B], adjb[B])
            if b == 0:
                pl.when(p > 0)(lambda: scatter_wait(2))
            else:
                scatter_wait(b - 1)
            idx_wait((b + 2) % 3)
            gather_issue((b + 2) % 3)
            if b == 2:
                pl.when(p < NT - 1)(lambda: idx_issue(c + 3, B))
            else:
                idx_issue(c + 3, B)
            scatter_issue(c, B)
        return carry

    lax.fori_loop(0, NT, _triple, 0)

    gather_wait(0)
    _scale_chunk(rows[0], adjb[0])
    scatter_wait(2)
    scatter_issue(NCH - 2, 0)
    gather_wait(1)
    _scale_chunk(rows[1], adjb[1])
    scatter_wait(0)
    scatter_issue(NCH - 1, 1)
    scatter_wait(1)
    plsc.subcore_barrier()

    pltpu.sync_copy(acc_sh.at[pl.ds(sid * RPT, RPT)],
                    out_hbm.at[cid, pl.ds(sid * RPT, RPT)])

    @pl.when(sid == 0)
    def _dump_tail():
        pltpu.sync_copy(acc_sh.at[pl.ds(NS * RPT, TAIL)],
                        out_hbm.at[cid, pl.ds(NS * RPT, TAIL)])


def _sc_scatter(support, src, dst2d, adj):
    mesh = plsc.VectorSubcoreMesh(core_axis_name="c", subcore_axis_name="s")
    k = pl.kernel(
        _sc_body,
        mesh=mesh,
        out_type=jax.ShapeDtypeStruct((NC, N, F), jnp.float32),
        scratch_types=[
            pltpu.VMEM((NCH, CH), jnp.int32),
            pltpu.VMEM((CH, F), jnp.float32),
            pltpu.VMEM((CH, F), jnp.float32),
            pltpu.VMEM((CH, F), jnp.float32),
            pltpu.VMEM((CH,), jnp.int32),
            pltpu.VMEM((CH,), jnp.int32),
            pltpu.VMEM((CH,), jnp.int32),
            pltpu.VMEM((CH,), jnp.float32),
            pltpu.VMEM((CH,), jnp.float32),
            pltpu.VMEM((CH,), jnp.float32),
            pltpu.VMEM((ZR, F), jnp.float32),
            pltpu.VMEM_SHARED((N, F), jnp.float32),
            pltpu.SemaphoreType.DMA,
            pltpu.SemaphoreType.DMA,
            pltpu.SemaphoreType.DMA,
            pltpu.SemaphoreType.DMA,
            pltpu.SemaphoreType.DMA,
            pltpu.SemaphoreType.DMA,
            pltpu.SemaphoreType.DMA,
            pltpu.SemaphoreType.DMA,
            pltpu.SemaphoreType.DMA,
        ],
    )
    return k(support, src, dst2d, adj)


def _comb_body(p_ref, b_ref, o_ref):
    o_ref[...] = jnp.maximum(p_ref[0] + p_ref[1] + b_ref[...], 0.0)


def _combine(partials, b2d):
    blk = 1000
    return pl.pallas_call(
        _comb_body,
        grid=(N // blk,),
        in_specs=[
            pl.BlockSpec((NC, blk, F), lambda i: (0, i, 0)),
            pl.BlockSpec((1, F), lambda i: (0, 0)),
        ],
        out_specs=pl.BlockSpec((blk, F), lambda i: (i, 0)),
        out_shape=jax.ShapeDtypeStruct((N, F), jnp.float32),
    )(partials, b2d)


def kernel(x, edge_index, adj_values, W, b):
    support = _matmul(x, W)
    src = edge_index[0]
    dst2d = edge_index[1].reshape(NW, NCH, CH)
    partials = _sc_scatter(support, src, dst2d, adj_values)
    return _combine(partials, b.reshape(1, F))


# speedup vs baseline: 1.0037x; 1.0037x over previous
import functools

import jax
import jax.numpy as jnp
from jax import lax
from jax.experimental import pallas as pl
from jax.experimental.pallas import tpu as pltpu
from jax.experimental.pallas import tpu_sc as plsc

N = 10000
E = 320000
F = 128

NC = 2
NS = 16
NW = NC * NS
EPW = E // NW
CH = 80
NCH = EPW // CH
RPT = 624
TAIL = N - NS * RPT
ZR = 16
LN = 16


def _mm_body(x_ref, w_ref, o_ref):
    o_ref[...] = jnp.dot(x_ref[...], w_ref[...],
                         preferred_element_type=jnp.float32)


def _matmul(x, W):
    blk = 1000
    return pl.pallas_call(
        _mm_body,
        grid=(N // blk,),
        in_specs=[
            pl.BlockSpec((blk, F), lambda i: (i, 0)),
            pl.BlockSpec((F, F), lambda i: (0, 0)),
        ],
        out_specs=pl.BlockSpec((blk, F), lambda i: (i, 0)),
        out_shape=jax.ShapeDtypeStruct((N, F), jnp.float32),
    )(x, W)


def _scale_chunk(rows, adjb):
    for go in range(0, CH, LN):
        a16 = adjb[pl.ds(go, LN)]
        for j in range(LN):
            for s in range(F // LN):
                sl = pl.ds(s * LN, LN)
                rows[go + j, sl] = rows[go + j, sl] * a16[j]


def _sc_body(support_hbm, src_hbm, dst_hbm, adj_hbm, out_hbm,
             dst_v, rows0_v, rows1_v, rows2_v,
             srcb0_v, srcb1_v, srcb2_v, adjb0_v, adjb1_v, adjb2_v,
             zbuf_v, acc_sh,
             gsem0, gsem1, gsem2, ssem0, ssem1, ssem2, isem0, isem1, isem2):
    cid = lax.axis_index("c")
    sid = lax.axis_index("s")
    wid = cid * NS + sid
    ebase = wid * EPW

    pltpu.sync_copy(dst_hbm.at[wid], dst_v)

    zeros = jnp.zeros((LN,), jnp.float32)

    def _zrow(i, carry):
        for s in range(F // LN):
            zbuf_v[i, pl.ds(s * LN, LN)] = zeros
        return carry

    lax.fori_loop(0, ZR, _zrow, 0)
    for r in range(RPT // ZR):
        pltpu.sync_copy(zbuf_v, acc_sh.at[pl.ds(sid * RPT + r * ZR, ZR)])

    @pl.when(sid == 0)
    def _zero_tail():
        pltpu.sync_copy(zbuf_v.at[pl.ds(0, TAIL)],
                        acc_sh.at[pl.ds(NS * RPT, TAIL)])

    plsc.subcore_barrier()

    rows = (rows0_v, rows1_v, rows2_v)
    srcb = (srcb0_v, srcb1_v, srcb2_v)
    adjb = (adjb0_v, adjb1_v, adjb2_v)
    gsem = (gsem0, gsem1, gsem2)
    ssem = (ssem0, ssem1, ssem2)
    isem = (isem0, isem1, isem2)

    def idx_issue(c, B):
        pltpu.async_copy(src_hbm.at[pl.ds(ebase + c * CH, CH)], srcb[B],
                         isem[B])
        pltpu.async_copy(adj_hbm.at[pl.ds(ebase + c * CH, CH)], adjb[B],
                         isem[B])

    def idx_wait(B):
        pltpu.make_async_copy(src_hbm.at[pl.ds(0, CH)], srcb[B],
                              isem[B]).wait()
        pltpu.make_async_copy(adj_hbm.at[pl.ds(0, CH)], adjb[B],
                              isem[B]).wait()

    def gather_issue(B):
        h = CH // 2
        pltpu.async_copy(support_hbm.at[srcb[B].at[pl.ds(0, h)]],
                         rows[B].at[pl.ds(0, h)], gsem[B])
        pltpu.async_copy(support_hbm.at[srcb[B].at[pl.ds(h, h)]],
                         rows[B].at[pl.ds(h, h)], gsem[B])

    def gather_wait(B):
        pltpu.make_async_copy(support_hbm.at[pl.ds(0, CH)], rows[B],
                              gsem[B]).wait()

    def scatter_issue(c, B):
        pltpu.async_copy(rows[B], acc_sh.at[dst_v.at[c]], ssem[B], add=True)

    def scatter_wait(B):
        pltpu.make_async_copy(support_hbm.at[pl.ds(0, CH)], rows[B],
                              ssem[B]).wait()

    idx_issue(0, 0)
    idx_issue(1, 1)
    idx_issue(2, 2)
    idx_wait(0)
    gather_issue(0)
    idx_wait(1)
    gather_issue(1)

    NT = (NCH - 2) // 3

    def _triple(p, carry):
        for b in range(3):
            c = 3 * p + b
            B = b
            gather_wait(B)
            _scale_chunk(rows[B], adjb[B])
            if b == 0:
                pl.when(p > 0)(lambda: scatter_wait(2))
            else:
                scatter_wait(b - 1)
            idx_wait((b + 2) % 3)
            gather_issue((b + 2) % 3)
            if b == 2:
                pl.when(p < NT - 1)(lambda: idx_issue(c + 3, B))
            else:
                idx_issue(c + 3, B)
            scatter_issue(c, B)
        return carry

    lax.fori_loop(0, NT, _triple, 0)

    gather_wait(0)
    _scale_chunk(rows[0], adjb[0])
    scatter_wait(2)
    scatter_issue(NCH - 2, 0)
    gather_wait(1)
    _scale_chunk(rows[1], adjb[1])
    scatter_wait(0)
    scatter_issue(NCH - 1, 1)
    scatter_wait(1)
    plsc.subcore_barrier()

    pltpu.sync_copy(acc_sh.at[pl.ds(sid * RPT, RPT)],
                    out_hbm.at[cid, pl.ds(sid * RPT, RPT)])

    @pl.when(sid == 0)
    def _dump_tail():
        pltpu.sync_copy(acc_sh.at[pl.ds(NS * RPT, TAIL)],
                        out_hbm.at[cid, pl.ds(NS * RPT, TAIL)])


def _sc_scatter(support, src, dst2d, adj):
    mesh = plsc.VectorSubcoreMesh(core_axis_name="c", subcore_axis_name="s")
    k = pl.kernel(
        _sc_body,
        mesh=mesh,
        out_type=jax.ShapeDtypeStruct((NC, N, F), jnp.float32),
        scratch_types=[
            pltpu.VMEM((NCH, CH), jnp.int32),
            pltpu.VMEM((CH, F), jnp.float32),
            pltpu.VMEM((CH, F), jnp.float32),
            pltpu.VMEM((CH, F), jnp.float32),
            pltpu.VMEM((CH,), jnp.int32),
            pltpu.VMEM((CH,), jnp.int32),
            pltpu.VMEM((CH,), jnp.int32),
            pltpu.VMEM((CH,), jnp.float32),
            pltpu.VMEM((CH,), jnp.float32),
            pltpu.VMEM((CH,), jnp.float32),
            pltpu.VMEM((ZR, F), jnp.float32),
            pltpu.VMEM_SHARED((N, F), jnp.float32),
            pltpu.SemaphoreType.DMA,
            pltpu.SemaphoreType.DMA,
            pltpu.SemaphoreType.DMA,
            pltpu.SemaphoreType.DMA,
            pltpu.SemaphoreType.DMA,
            pltpu.SemaphoreType.DMA,
            pltpu.SemaphoreType.DMA,
            pltpu.SemaphoreType.DMA,
            pltpu.SemaphoreType.DMA,
        ],
    )
    return k(support, src, dst2d, adj)


def _comb_body(p_ref, b_ref, o_ref):
    o_ref[...] = jnp.maximum(p_ref[0] + p_ref[1] + b_ref[...], 0.0)


def _combine(partials, b2d):
    blk = 1000
    return pl.pallas_call(
        _comb_body,
        grid=(N // blk,),
        in_specs=[
            pl.BlockSpec((NC, blk, F), lambda i: (0, i, 0)),
            pl.BlockSpec((1, F), lambda i: (0, 0)),
        ],
        out_specs=pl.BlockSpec((blk, F), lambda i: (i, 0)),
        out_shape=jax.ShapeDtypeStruct((N, F), jnp.float32),
    )(partials, b2d)


def kernel(x, edge_index, adj_values, W, b):
    support = _matmul(x, W)
    src = edge_index[0]
    dst2d = edge_index[1].reshape(NW, NCH, CH)
    partials = _sc_scatter(support, src, dst2d, adj_values)
    return _combine(partials, b.reshape(1, F))


# speedup vs baseline: 1.0152x; 1.0114x over previous
import functools

import jax
import jax.numpy as jnp
from jax import lax
from jax.experimental import pallas as pl
from jax.experimental.pallas import tpu as pltpu
from jax.experimental.pallas import tpu_sc as plsc

N = 10000
E = 320000
F = 128

NC = 2
NS = 16
NW = NC * NS
EPW = E // NW
CH = 80
NCH = EPW // CH
RPT = 624
TAIL = N - NS * RPT
ZR = 16
LN = 16


def _mm_body(x_ref, w_ref, o_ref):
    o_ref[...] = jnp.dot(x_ref[...], w_ref[...],
                         preferred_element_type=jnp.float32)


def _matmul(x, W):
    blk = 1000
    return pl.pallas_call(
        _mm_body,
        grid=(N // blk,),
        in_specs=[
            pl.BlockSpec((blk, F), lambda i: (i, 0)),
            pl.BlockSpec((F, F), lambda i: (0, 0)),
        ],
        out_specs=pl.BlockSpec((blk, F), lambda i: (i, 0)),
        out_shape=jax.ShapeDtypeStruct((N, F), jnp.float32),
    )(x, W)


def _scale_chunk(rows, adjb):
    for go in range(0, CH, LN):
        a16 = adjb[pl.ds(go, LN)]
        for j in range(LN):
            for s in range(F // LN):
                sl = pl.ds(s * LN, LN)
                rows[go + j, sl] = rows[go + j, sl] * a16[j]


def _sc_body(support_hbm, src_hbm, dst_hbm, adj_hbm, out_hbm,
             dst_v, rows0_v, rows1_v, rows2_v,
             srcb0_v, srcb1_v, srcb2_v, adjb0_v, adjb1_v, adjb2_v,
             zbuf_v, acc_sh,
             gsem0, gsem1, gsem2, ssem0, ssem1, ssem2, isem0, isem1, isem2,
             zsem):
    cid = lax.axis_index("c")
    sid = lax.axis_index("s")
    wid = cid * NS + sid
    ebase = wid * EPW

    pltpu.sync_copy(dst_hbm.at[wid], dst_v)

    zeros = jnp.zeros((LN,), jnp.float32)

    def _zrow(i, carry):
        for s in range(F // LN):
            zbuf_v[i, pl.ds(s * LN, LN)] = zeros
        return carry

    lax.fori_loop(0, ZR, _zrow, 0)
    for r in range(RPT // ZR):
        pltpu.async_copy(zbuf_v, acc_sh.at[pl.ds(sid * RPT + r * ZR, ZR)],
                         zsem)

    @pl.when(sid == 0)
    def _zero_tail():
        pltpu.async_copy(zbuf_v.at[pl.ds(0, TAIL)],
                         acc_sh.at[pl.ds(NS * RPT, TAIL)], zsem)

    rows = (rows0_v, rows1_v, rows2_v)
    srcb = (srcb0_v, srcb1_v, srcb2_v)
    adjb = (adjb0_v, adjb1_v, adjb2_v)
    gsem = (gsem0, gsem1, gsem2)
    ssem = (ssem0, ssem1, ssem2)
    isem = (isem0, isem1, isem2)

    def idx_issue(c, B):
        pltpu.async_copy(src_hbm.at[pl.ds(ebase + c * CH, CH)], srcb[B],
                         isem[B])
        pltpu.async_copy(adj_hbm.at[pl.ds(ebase + c * CH, CH)], adjb[B],
                         isem[B])

    def idx_wait(B):
        pltpu.make_async_copy(src_hbm.at[pl.ds(0, CH)], srcb[B],
                              isem[B]).wait()
        pltpu.make_async_copy(adj_hbm.at[pl.ds(0, CH)], adjb[B],
                              isem[B]).wait()

    def gather_issue(B):
        h = CH // 2
        pltpu.async_copy(support_hbm.at[srcb[B].at[pl.ds(0, h)]],
                         rows[B].at[pl.ds(0, h)], gsem[B])
        pltpu.async_copy(support_hbm.at[srcb[B].at[pl.ds(h, h)]],
                         rows[B].at[pl.ds(h, h)], gsem[B])

    def gather_wait(B):
        pltpu.make_async_copy(support_hbm.at[pl.ds(0, CH)], rows[B],
                              gsem[B]).wait()

    def scatter_issue(c, B):
        pltpu.async_copy(rows[B], acc_sh.at[dst_v.at[c]], ssem[B], add=True)

    def scatter_wait(B):
        pltpu.make_async_copy(support_hbm.at[pl.ds(0, CH)], rows[B],
                              ssem[B]).wait()

    idx_issue(0, 0)
    idx_issue(1, 1)
    idx_issue(2, 2)
    idx_wait(0)
    gather_issue(0)
    idx_wait(1)
    gather_issue(1)

    for r in range(RPT // ZR):
        pltpu.make_async_copy(zbuf_v, acc_sh.at[pl.ds(0, ZR)], zsem).wait()

    @pl.when(sid == 0)
    def _drain_tail():
        pltpu.make_async_copy(zbuf_v.at[pl.ds(0, TAIL)],
                              acc_sh.at[pl.ds(0, TAIL)], zsem).wait()

    plsc.subcore_barrier()

    NT = (NCH - 2) // 3

    def _triple(p, carry):
        for b in range(3):
            c = 3 * p + b
            B = b
            gather_wait(B)
            _scale_chunk(rows[B], adjb[B])
            if b == 0:
                pl.when(p > 0)(lambda: scatter_wait(2))
            else:
                scatter_wait(b - 1)
            idx_wait((b + 2) % 3)
            gather_issue((b + 2) % 3)
            if b == 2:
                pl.when(p < NT - 1)(lambda: idx_issue(c + 3, B))
            else:
                idx_issue(c + 3, B)
            scatter_issue(c, B)
        return carry

    lax.fori_loop(0, NT, _triple, 0)

    gather_wait(0)
    _scale_chunk(rows[0], adjb[0])
    scatter_wait(2)
    scatter_issue(NCH - 2, 0)
    gather_wait(1)
    _scale_chunk(rows[1], adjb[1])
    scatter_wait(0)
    scatter_issue(NCH - 1, 1)
    scatter_wait(1)
    plsc.subcore_barrier()

    pltpu.sync_copy(acc_sh.at[pl.ds(sid * RPT, RPT)],
                    out_hbm.at[cid, pl.ds(sid * RPT, RPT)])

    @pl.when(sid == 0)
    def _dump_tail():
        pltpu.sync_copy(acc_sh.at[pl.ds(NS * RPT, TAIL)],
                        out_hbm.at[cid, pl.ds(NS * RPT, TAIL)])


def _sc_scatter(support, src, dst2d, adj):
    mesh = plsc.VectorSubcoreMesh(core_axis_name="c", subcore_axis_name="s")
    k = pl.kernel(
        _sc_body,
        mesh=mesh,
        out_type=jax.ShapeDtypeStruct((NC, N, F), jnp.float32),
        scratch_types=[
            pltpu.VMEM((NCH, CH), jnp.int32),
            pltpu.VMEM((CH, F), jnp.float32),
            pltpu.VMEM((CH, F), jnp.float32),
            pltpu.VMEM((CH, F), jnp.float32),
            pltpu.VMEM((CH,), jnp.int32),
            pltpu.VMEM((CH,), jnp.int32),
            pltpu.VMEM((CH,), jnp.int32),
            pltpu.VMEM((CH,), jnp.float32),
            pltpu.VMEM((CH,), jnp.float32),
            pltpu.VMEM((CH,), jnp.float32),
            pltpu.VMEM((ZR, F), jnp.float32),
            pltpu.VMEM_SHARED((N, F), jnp.float32),
            pltpu.SemaphoreType.DMA,
            pltpu.SemaphoreType.DMA,
            pltpu.SemaphoreType.DMA,
            pltpu.SemaphoreType.DMA,
            pltpu.SemaphoreType.DMA,
            pltpu.SemaphoreType.DMA,
            pltpu.SemaphoreType.DMA,
            pltpu.SemaphoreType.DMA,
            pltpu.SemaphoreType.DMA,
            pltpu.SemaphoreType.DMA,
        ],
    )
    return k(support, src, dst2d, adj)


def _comb_body(p_ref, b_ref, o_ref):
    o_ref[...] = jnp.maximum(p_ref[0] + p_ref[1] + b_ref[...], 0.0)


def _combine(partials, b2d):
    blk = 1000
    return pl.pallas_call(
        _comb_body,
        grid=(N // blk,),
        in_specs=[
            pl.BlockSpec((NC, blk, F), lambda i: (0, i, 0)),
            pl.BlockSpec((1, F), lambda i: (0, 0)),
        ],
        out_specs=pl.BlockSpec((blk, F), lambda i: (i, 0)),
        out_shape=jax.ShapeDtypeStruct((N, F), jnp.float32),
    )(partials, b2d)


def kernel(x, edge_index, adj_values, W, b):
    support = _matmul(x, W)
    src = edge_index[0]
    dst2d = edge_index[1].reshape(NW, NCH, CH)
    partials = _sc_scatter(support, src, dst2d, adj_values)
    return _combine(partials, b.reshape(1, F))
